# Initial kernel scaffold; baseline (speedup 1.0000x reference)
import jax
import jax.numpy as jnp
from jax.experimental import pallas as pl


def kernel(x, t, pe):
    raise NotImplementedError("write your pallas kernel here")


# speedup vs baseline: 1.9587x; 1.9587x over previous
import functools

import jax
import jax.numpy as jnp
from jax.experimental import pallas as pl
from jax.experimental.pallas import tpu as pltpu
from jax.experimental.pallas import tpu_sc as plsc

_D = 64
_W = 128
_LANES = 16


def _sc_call(x2, t2, pe):
    n, d = x2.shape

    @functools.partial(
        pl.kernel,
        out_type=jax.ShapeDtypeStruct((n, d), jnp.float32),
        mesh=plsc.VectorSubcoreMesh(core_axis_name="c", subcore_axis_name="s"),
        scratch_types=[pltpu.VMEM((_W, d), jnp.float32)],
        compiler_params=pltpu.CompilerParams(use_tc_tiling_on_sc=False),
    )
    def sc_kernel(x_hbm, t_hbm, pe_hbm, o_hbm, g_vmem):
        def body(t_vmem, x_vmem, o_vmem):
            pltpu.sync_copy(pe_hbm.at[t_vmem.at[0]], g_vmem)

            @pl.loop(0, _W)
            def _row(r):
                for c in range(d // _LANES):
                    slc = (pl.ds(r, 1), pl.ds(c * _LANES, _LANES))
                    o_vmem.at[*slc][...] = (
                        x_vmem.at[*slc][...] + g_vmem.at[*slc][...]
                    )

        pltpu.emit_pipeline(
            body,
            grid=(n // _W,),
            in_specs=[
                pl.BlockSpec((1, _W), lambda i: (0, i)),
                pl.BlockSpec((_W, d), lambda i: (i, 0)),
            ],
            out_specs=[pl.BlockSpec((_W, d), lambda i: (i, 0))],
            core_axis_name=("c", "s"),
            dimension_semantics=(pltpu.PARALLEL,),
        )(t_hbm, x_hbm, o_hbm)

    return sc_kernel(x2, t2, pe)


@jax.jit
def kernel(x, t, pe):
    b, h, d = x.shape
    n = b * h
    x2 = x.reshape(n, d)
    t2 = t.reshape(1, n).astype(jnp.int32)
    out = _sc_call(x2, t2, pe)
    return out.reshape(b, h, d)


# speedup vs baseline: 2.1408x; 1.0930x over previous
import functools

import jax
import jax.numpy as jnp
from jax.experimental import pallas as pl
from jax.experimental.pallas import tpu as pltpu
from jax.experimental.pallas import tpu_sc as plsc

_D = 64
_W = 128
_LANES = 16


def _sc_call(x2, t2, pe):
    n, d = x2.shape

    n_pe = pe.shape[0]
    pe_rows_per_subcore = n_pe // 16

    @functools.partial(
        pl.kernel,
        out_type=jax.ShapeDtypeStruct((n, d), jnp.float32),
        mesh=plsc.VectorSubcoreMesh(core_axis_name="c", subcore_axis_name="s"),
        scratch_types=[
            pltpu.VMEM((_W, d), jnp.float32),
            pltpu.VMEM_SHARED((n_pe, d), jnp.float32),
        ],
        compiler_params=pltpu.CompilerParams(use_tc_tiling_on_sc=False),
    )
    def sc_kernel(x_hbm, t_hbm, pe_hbm, o_hbm, g_vmem, pe_sh):
        s = jax.lax.axis_index("s")
        row0 = s * pe_rows_per_subcore
        pltpu.sync_copy(
            pe_hbm.at[pl.ds(row0, pe_rows_per_subcore)],
            pe_sh.at[pl.ds(row0, pe_rows_per_subcore)],
        )
        plsc.subcore_barrier()

        def body(t_vmem, x_vmem, o_vmem):
            pltpu.sync_copy(pe_sh.at[t_vmem.at[0]], g_vmem)

            @pl.loop(0, _W)
            def _row(r):
                for c in range(d // _LANES):
                    slc = (pl.ds(r, 1), pl.ds(c * _LANES, _LANES))
                    o_vmem.at[*slc][...] = (
                        x_vmem.at[*slc][...] + g_vmem.at[*slc][...]
                    )

        pltpu.emit_pipeline(
            body,
            grid=(n // _W,),
            in_specs=[
                pl.BlockSpec((1, _W), lambda i: (0, i)),
                pl.BlockSpec((_W, d), lambda i: (i, 0)),
            ],
            out_specs=[pl.BlockSpec((_W, d), lambda i: (i, 0))],
            core_axis_name=("c", "s"),
            dimension_semantics=(pltpu.PARALLEL,),
        )(t_hbm, x_hbm, o_hbm)

    return sc_kernel(x2, t2, pe)


@jax.jit
def kernel(x, t, pe):
    b, h, d = x.shape
    n = b * h
    x2 = x.reshape(n, d)
    t2 = t.reshape(1, n).astype(jnp.int32)
    out = _sc_call(x2, t2, pe)
    return out.reshape(b, h, d)


# speedup vs baseline: 2.3034x; 1.0760x over previous
import functools

import jax
import jax.numpy as jnp
from jax.experimental import pallas as pl
from jax.experimental.pallas import tpu as pltpu
from jax.experimental.pallas import tpu_sc as plsc

_D = 64
_W = 128
_LANES = 16


def _sc_call(x2, t2, pe):
    n, d = x2.shape

    n_pe = pe.shape[0]
    pe_rows_per_subcore = n_pe // 16

    @functools.partial(
        pl.kernel,
        out_type=jax.ShapeDtypeStruct((n, d), jnp.float32),
        mesh=plsc.VectorSubcoreMesh(core_axis_name="c", subcore_axis_name="s"),
        scratch_types=[
            pltpu.VMEM_SHARED((n_pe, d), jnp.float32),
        ],
        compiler_params=pltpu.CompilerParams(use_tc_tiling_on_sc=False),
    )
    def sc_kernel(x_hbm, t_hbm, pe_hbm, o_hbm, pe_sh):
        s = jax.lax.axis_index("s")
        row0 = s * pe_rows_per_subcore
        pltpu.sync_copy(
            pe_hbm.at[pl.ds(row0, pe_rows_per_subcore)],
            pe_sh.at[pl.ds(row0, pe_rows_per_subcore)],
        )
        plsc.subcore_barrier()

        def body(t_vmem, x_vmem, o_vmem):
            pltpu.sync_copy(pe_sh.at[t_vmem.at[0]], o_vmem)

            @pl.loop(0, _W)
            def _row(r):
                for c in range(d // _LANES):
                    slc = (pl.ds(r, 1), pl.ds(c * _LANES, _LANES))
                    plsc.addupdate(o_vmem.at[*slc], x_vmem.at[*slc][...])

        pltpu.emit_pipeline(
            body,
            grid=(n // _W,),
            in_specs=[
                pl.BlockSpec((1, _W), lambda i: (0, i)),
                pl.BlockSpec((_W, d), lambda i: (i, 0)),
            ],
            out_specs=[pl.BlockSpec((_W, d), lambda i: (i, 0))],
            core_axis_name=("c", "s"),
            dimension_semantics=(pltpu.PARALLEL,),
        )(t_hbm, x_hbm, o_hbm)

    return sc_kernel(x2, t2, pe)


@jax.jit
def kernel(x, t, pe):
    b, h, d = x.shape
    n = b * h
    x2 = x.reshape(n, d)
    t2 = t.reshape(1, n).astype(jnp.int32)
    out = _sc_call(x2, t2, pe)
    return out.reshape(b, h, d)


# speedup vs baseline: 2.8872x; 1.2534x over previous
import functools

import jax
import jax.numpy as jnp
from jax.experimental import pallas as pl
from jax.experimental.pallas import tpu as pltpu
from jax.experimental.pallas import tpu_sc as plsc

_D = 64
_LANES = 16
_NC = 2
_NS = 16
_NW = _NC * _NS
_C = 128
_GW = 128


def _sc_call(x2, t1, pe):
    n, d = x2.shape
    n_pe = pe.shape[0]
    rpw = n // _NW
    nch = rpw // _C
    pe_rows_per_subcore = n_pe // _NS

    @functools.partial(
        pl.kernel,
        out_type=jax.ShapeDtypeStruct((n, d), jnp.float32),
        mesh=plsc.VectorSubcoreMesh(core_axis_name="c", subcore_axis_name="s"),
        scratch_types=[
            pltpu.VMEM_SHARED((n_pe, d), jnp.float32),
            pltpu.VMEM((rpw,), jnp.int32),
            pltpu.VMEM((_C, d), jnp.float32),
            pltpu.VMEM((_C, d), jnp.float32),
            pltpu.VMEM((_C, d), jnp.float32),
            pltpu.VMEM((_C, d), jnp.float32),
            pltpu.SemaphoreType.DMA,
            pltpu.SemaphoreType.DMA,
            pltpu.SemaphoreType.DMA,
            pltpu.SemaphoreType.DMA,
            pltpu.SemaphoreType.DMA,
            pltpu.SemaphoreType.DMA,
        ],
        compiler_params=pltpu.CompilerParams(use_tc_tiling_on_sc=False),
    )
    def sc_kernel(x_hbm, t_hbm, pe_hbm, o_hbm, pe_sh, idx_v,
                  xb0, xb1, gb0, gb1, sx0, sx1, sg0, sg1, so0, so1):
        c_idx = jax.lax.axis_index("c")
        s_idx = jax.lax.axis_index("s")
        wid = c_idx * _NS + s_idx
        base = wid * rpw

        row0 = s_idx * pe_rows_per_subcore
        pltpu.sync_copy(
            pe_hbm.at[pl.ds(row0, pe_rows_per_subcore)],
            pe_sh.at[pl.ds(row0, pe_rows_per_subcore)],
        )
        pltpu.sync_copy(t_hbm.at[pl.ds(base, rpw)], idx_v)
        plsc.subcore_barrier()

        xb = (xb0, xb1)
        gb = (gb0, gb1)
        sx = (sx0, sx1)
        sg = (sg0, sg1)
        so = (so0, so1)

        def x_copy(i, p):
            return pltpu.make_async_copy(
                x_hbm.at[pl.ds(base + i * _C, _C)], xb[p], sx[p])

        def g_copy(i, p, k):
            return pltpu.make_async_copy(
                pe_sh.at[idx_v.at[pl.ds(i * _C + k * _GW, _GW)]],
                gb[p].at[pl.ds(k * _GW, _GW)],
                sg[p])

        def o_copy(i, p):
            return pltpu.make_async_copy(
                gb[p], o_hbm.at[pl.ds(base + i * _C, _C)], so[p])

        def start_chunk(i, p):
            x_copy(i, p).start()
            for k in range(_C // _GW):
                g_copy(i, p, k).start()

        def wait_chunk(i, p):
            x_copy(i, p).wait()
            for k in range(_C // _GW):
                g_copy(i, p, k).wait()

        start_chunk(0, 0)

        @pl.loop(0, nch, step=2)
        def _outer(i0):
            for p in (0, 1):
                i = i0 + p
                q = 1 - p

                @pl.when(i + 1 < nch)
                def _prefetch():
                    @pl.when(i >= 1)
                    def _drain_prev_out():
                        o_copy(i - 1, q).wait()

                    start_chunk(i + 1, q)

                wait_chunk(i, p)

                @pl.loop(0, _C)
                def _row(r):
                    for cc in range(d // _LANES):
                        slc = (pl.ds(r, 1), pl.ds(cc * _LANES, _LANES))
                        plsc.addupdate(gb[p].at[*slc], xb[p].at[*slc][...])

                o_copy(i, p).start()

        o_copy(nch - 2, 0).wait()
        o_copy(nch - 1, 1).wait()

    return sc_kernel(x2, t1, pe)


@jax.jit
def kernel(x, t, pe):
    b, h, d = x.shape
    n = b * h
    x2 = x.reshape(n, d)
    t1 = t.reshape(n).astype(jnp.int32)
    out = _sc_call(x2, t1, pe)
    return out.reshape(b, h, d)


# speedup vs baseline: 2.9533x; 1.0229x over previous
import functools

import jax
import jax.numpy as jnp
from jax.experimental import pallas as pl
from jax.experimental.pallas import tpu as pltpu
from jax.experimental.pallas import tpu_sc as plsc

_LANES = 16
_NC = 2
_NS = 16
_NW = _NC * _NS
_GW = 128


def _sc_call(x, t1, pe):
    nb, nh, d = x.shape
    n_pe = pe.shape[0]
    bpw = nb // _NW
    rpw = bpw * nh
    pe_rows_per_subcore = n_pe // _NS

    @functools.partial(
        pl.kernel,
        out_type=jax.ShapeDtypeStruct((nb, nh, d), jnp.float32),
        mesh=plsc.VectorSubcoreMesh(core_axis_name="c", subcore_axis_name="s"),
        scratch_types=[
            pltpu.VMEM_SHARED((n_pe, d), jnp.float32),
            pltpu.VMEM((rpw,), jnp.int32),
            pltpu.VMEM((nh, d), jnp.float32),
            pltpu.VMEM((nh, d), jnp.float32),
            pltpu.VMEM((nh, d), jnp.float32),
            pltpu.VMEM((nh, d), jnp.float32),
            pltpu.SemaphoreType.DMA,
            pltpu.SemaphoreType.DMA,
            pltpu.SemaphoreType.DMA,
            pltpu.SemaphoreType.DMA,
            pltpu.SemaphoreType.DMA,
            pltpu.SemaphoreType.DMA,
        ],
        compiler_params=pltpu.CompilerParams(use_tc_tiling_on_sc=False),
    )
    def sc_kernel(x_hbm, t_hbm, pe_hbm, o_hbm, pe_sh, idx_v,
                  xb0, xb1, gb0, gb1, sx0, sx1, sg0, sg1, so0, so1):
        c_idx = jax.lax.axis_index("c")
        s_idx = jax.lax.axis_index("s")
        wid = c_idx * _NS + s_idx
        base_b = wid * bpw

        row0 = s_idx * pe_rows_per_subcore
        pltpu.sync_copy(
            pe_hbm.at[pl.ds(row0, pe_rows_per_subcore)],
            pe_sh.at[pl.ds(row0, pe_rows_per_subcore)],
        )
        pltpu.sync_copy(t_hbm.at[pl.ds(base_b * nh, rpw)], idx_v)
        plsc.subcore_barrier()

        xb = (xb0, xb1)
        gb = (gb0, gb1)
        sx = (sx0, sx1)
        sg = (sg0, sg1)
        so = (so0, so1)

        g_splits = []
        off = 0
        while off < nh:
            g_splits.append((off, min(_GW, nh - off)))
            off += _GW

        def x_copy(i, p):
            return pltpu.make_async_copy(x_hbm.at[base_b + i], xb[p], sx[p])

        def g_copy(i, p, off, cnt):
            return pltpu.make_async_copy(
                pe_sh.at[idx_v.at[pl.ds(i * nh + off, cnt)]],
                gb[p].at[pl.ds(off, cnt)],
                sg[p])

        def o_copy(i, p):
            return pltpu.make_async_copy(gb[p], o_hbm.at[base_b + i], so[p])

        def start_chunk(i, p):
            x_copy(i, p).start()
            for off, cnt in g_splits:
                g_copy(i, p, off, cnt).start()

        def wait_chunk(i, p):
            x_copy(i, p).wait()
            for off, cnt in g_splits:
                g_copy(i, p, off, cnt).wait()

        start_chunk(0, 0)

        @pl.loop(0, bpw, step=2)
        def _outer(i0):
            for p in (0, 1):
                i = i0 + p
                q = 1 - p

                @pl.when(i + 1 < bpw)
                def _prefetch():
                    @pl.when(i >= 1)
                    def _drain_prev_out():
                        o_copy(i - 1, q).wait()

                    start_chunk(i + 1, q)

                wait_chunk(i, p)

                @pl.loop(0, nh)
                def _row(r):
                    for cc in range(d // _LANES):
                        slc = (pl.ds(r, 1), pl.ds(cc * _LANES, _LANES))
                        plsc.addupdate(gb[p].at[*slc], xb[p].at[*slc][...])

                o_copy(i, p).start()

        o_copy(bpw - 2, 0).wait()
        o_copy(bpw - 1, 1).wait()

    return sc_kernel(x, t1, pe)


@jax.jit
def kernel(x, t, pe):
    t1 = t.reshape(t.shape[0] * t.shape[1]).astype(jnp.int32)
    return _sc_call(x, t1, pe)


# speedup vs baseline: 3.5645x; 1.2070x over previous
import dataclasses
import functools

import jax
import jax.numpy as jnp
from jax import lax
from jax.experimental import pallas as pl
from jax.experimental.pallas import tpu as pltpu
from jax.experimental.pallas import tpu_sc as plsc

_LANES = 16
_NC = 2
_NS = 16
_NW = _NC * _NS
_DG = 8
_TT = 79

_CP = pltpu.CompilerParams()
if "needs_layout_passes" in pltpu.CompilerParams.__dataclass_fields__:
    _CP = dataclasses.replace(_CP, needs_layout_passes=False)


def _sc_call(xP, tP, pe_flat):
    nh, d, nb = xP.shape
    ng = d // _DG
    nbr = _NW // ng
    bw = nb // nbr
    tbl = _TT * _DG * 128

    @functools.partial(
        pl.kernel,
        out_type=jax.ShapeDtypeStruct((nh, d, nb), jnp.float32),
        mesh=plsc.VectorSubcoreMesh(core_axis_name="c", subcore_axis_name="s"),
        scratch_types=[
            pltpu.VMEM((tbl,), jnp.float32),
            pltpu.VMEM((_DG, bw), jnp.float32),
            pltpu.VMEM((_DG, bw), jnp.float32),
            pltpu.VMEM((bw,), jnp.int32),
            pltpu.VMEM((bw,), jnp.int32),
            pltpu.SemaphoreType.DMA,
            pltpu.SemaphoreType.DMA,
            pltpu.SemaphoreType.DMA,
            pltpu.SemaphoreType.DMA,
            pltpu.SemaphoreType.DMA,
            pltpu.SemaphoreType.DMA,
        ],
        compiler_params=_CP,
    )
    def sc_kernel(x_hbm, t_hbm, pe_hbm, o_hbm, pv, xb0, xb1, ib0, ib1,
                  sx0, sx1, si0, si1, so0, so1):
        wid = lax.axis_index("c") * _NS + lax.axis_index("s")
        g = wid % ng
        d0 = g * _DG
        b0 = (wid // ng) * bw

        pltpu.sync_copy(pe_hbm.at[pl.ds(g * tbl, tbl)], pv)

        xb = (xb0, xb1)
        ib = (ib0, ib1)
        sx = (sx0, sx1)
        si = (si0, si1)
        so = (so0, so1)

        def x_copy(h, p):
            return pltpu.make_async_copy(
                x_hbm.at[h, pl.ds(d0, _DG), pl.ds(b0, bw)], xb[p], sx[p])

        def i_copy(h, p):
            return pltpu.make_async_copy(
                t_hbm.at[h, pl.ds(b0, bw)], ib[p], si[p])

        def o_copy(h, p):
            return pltpu.make_async_copy(
                xb[p], o_hbm.at[h, pl.ds(d0, _DG), pl.ds(b0, bw)], so[p])

        def start_chunk(h, p):
            x_copy(h, p).start()
            i_copy(h, p).start()

        def wait_chunk(h, p):
            x_copy(h, p).wait()
            i_copy(h, p).wait()

        start_chunk(0, 0)

        @pl.loop(0, nh, step=2)
        def _outer(h0):
            for p in (0, 1):
                h = h0 + p
                q = 1 - p

                @pl.when(h + 1 < nh)
                def _prefetch():
                    @pl.when(h >= 1)
                    def _drain_prev_out():
                        o_copy(h - 1, q).wait()

                    start_chunk(h + 1, q)

                wait_chunk(h, p)

                for j in range(bw // _LANES):
                    tv = ib[p][pl.ds(j * _LANES, _LANES)]
                    fb = ((tv >> 7) << 10) + (tv & 127)
                    for dd in range(_DG):
                        gv = plsc.load_gather(pv, [fb + dd * 128])
                        sl = (dd, pl.ds(j * _LANES, _LANES))
                        xb[p][sl] = xb[p][sl] + gv

                o_copy(h, p).start()

        o_copy(nh - 2, 0).wait()
        o_copy(nh - 1, 1).wait()

    return sc_kernel(xP, tP, pe_flat)


@jax.jit
def kernel(x, t, pe):
    n_pe, d = pe.shape
    xP = jnp.transpose(x, (1, 2, 0))
    tP = jnp.transpose(t, (1, 0)).astype(jnp.int32)
    peT = jnp.transpose(pe, (1, 0))
    pePad = jnp.pad(peT, ((0, 0), (0, _TT * 128 - n_pe)))
    pe_flat = (
        pePad.reshape(d // _DG, _DG, _TT, 128)
        .transpose(0, 2, 1, 3)
        .reshape(-1)
    )
    outP = _sc_call(xP, tP, pe_flat)
    return jnp.transpose(outP, (2, 0, 1))


# speedup vs baseline: 4.1193x; 1.1556x over previous
import dataclasses
import functools

import jax
import jax.numpy as jnp
from jax import lax
from jax.experimental import pallas as pl
from jax.experimental.pallas import tpu as pltpu
from jax.experimental.pallas import tpu_sc as plsc

_LANES = 16
_NC = 2
_NS = 16
_NW = _NC * _NS
_DG = 8
_TT = 79

_CP = pltpu.CompilerParams()
if "needs_layout_passes" in pltpu.CompilerParams.__dataclass_fields__:
    _CP = dataclasses.replace(_CP, needs_layout_passes=False)


def _sc_call(xP, tP, pe_flat):
    nh, d, nb = xP.shape
    ng = d // _DG
    nbr = _NW // ng
    bw = nb // nbr
    tbl = _TT * _DG * 128

    @functools.partial(
        pl.kernel,
        out_type=jax.ShapeDtypeStruct((nh, d, nb), jnp.float32),
        mesh=plsc.VectorSubcoreMesh(core_axis_name="c", subcore_axis_name="s"),
        scratch_types=[
            pltpu.VMEM((tbl,), jnp.float32),
            pltpu.VMEM((_DG, bw), jnp.float32),
            pltpu.VMEM((_DG, bw), jnp.float32),
            pltpu.VMEM((bw,), jnp.int32),
            pltpu.VMEM((bw,), jnp.int32),
            pltpu.SemaphoreType.DMA,
            pltpu.SemaphoreType.DMA,
            pltpu.SemaphoreType.DMA,
            pltpu.SemaphoreType.DMA,
            pltpu.SemaphoreType.DMA,
            pltpu.SemaphoreType.DMA,
        ],
        compiler_params=_CP,
    )
    def sc_kernel(x_hbm, t_hbm, pe_hbm, o_hbm, pv, xb0, xb1, ib0, ib1,
                  sx0, sx1, si0, si1, so0, so1):
        wid = lax.axis_index("c") * _NS + lax.axis_index("s")
        g = wid % ng
        d0 = g * _DG
        b0 = (wid // ng) * bw

        pltpu.sync_copy(pe_hbm.at[pl.ds(g * tbl, tbl)], pv)

        xb = (xb0, xb1)
        ib = (ib0, ib1)
        sx = (sx0, sx1)
        si = (si0, si1)
        so = (so0, so1)

        def x_copy(h, p):
            return pltpu.make_async_copy(
                x_hbm.at[h, pl.ds(d0, _DG), pl.ds(b0, bw)], xb[p], sx[p])

        def i_copy(h, p):
            return pltpu.make_async_copy(
                t_hbm.at[h, pl.ds(b0, bw)], ib[p], si[p])

        def o_copy(h, p):
            return pltpu.make_async_copy(
                xb[p], o_hbm.at[h, pl.ds(d0, _DG), pl.ds(b0, bw)], so[p])

        def start_chunk(h, p):
            x_copy(h, p).start()
            i_copy(h, p).start()

        def wait_chunk(h, p):
            x_copy(h, p).wait()
            i_copy(h, p).wait()

        start_chunk(0, 0)

        @pl.loop(0, nh, step=2)
        def _outer(h0):
            for p in (0, 1):
                h = h0 + p
                q = 1 - p

                @pl.when(h + 1 < nh)
                def _prefetch():
                    @pl.when(h >= 1)
                    def _drain_prev_out():
                        o_copy(h - 1, q).wait()

                    start_chunk(h + 1, q)

                wait_chunk(h, p)

                for j in range(bw // _LANES):
                    tv = ib[p][pl.ds(j * _LANES, _LANES)]
                    fb = ((tv >> 7) << 10) + (tv & 127)
                    for dd in range(_DG):
                        gv = plsc.load_gather(
                            pv.at[pl.ds(dd * 128, (_TT - 1) * 1024 + 128)], [fb])
                        plsc.addupdate(
                            xb[p].at[dd, pl.ds(j * _LANES, _LANES)], gv)

                o_copy(h, p).start()

        o_copy(nh - 2, 0).wait()
        o_copy(nh - 1, 1).wait()

    return sc_kernel(xP, tP, pe_flat)


@jax.jit
def kernel(x, t, pe):
    n_pe, d = pe.shape
    xP = jnp.transpose(x, (1, 2, 0))
    tP = jnp.transpose(t, (1, 0)).astype(jnp.int32)
    peT = jnp.transpose(pe, (1, 0))
    pePad = jnp.pad(peT, ((0, 0), (0, _TT * 128 - n_pe)))
    pe_flat = (
        pePad.reshape(d // _DG, _DG, _TT, 128)
        .transpose(0, 2, 1, 3)
        .reshape(-1)
    )
    outP = _sc_call(xP, tP, pe_flat)
    return jnp.transpose(outP, (2, 0, 1))


# speedup vs baseline: 13.1643x; 3.1957x over previous
import dataclasses
import functools

import jax
import jax.numpy as jnp
from jax import lax
from jax.experimental import pallas as pl
from jax.experimental.pallas import tpu as pltpu
from jax.experimental.pallas import tpu_sc as plsc

_LANES = 16
_NC = 2
_NS = 16
_NW = _NC * _NS
_DG = 8
_TT = 79

_CP = pltpu.CompilerParams()
if "needs_layout_passes" in pltpu.CompilerParams.__dataclass_fields__:
    _CP = dataclasses.replace(_CP, needs_layout_passes=False)


def _sc_call(xP, tP, pe_flat):
    nh, d, nb = xP.shape
    ng = d // _DG
    nbr = _NW // ng
    bw = nb // nbr
    tbl = _TT * _DG * 128

    @functools.partial(
        pl.kernel,
        out_type=jax.ShapeDtypeStruct((nh, d, nb), jnp.float32),
        mesh=plsc.VectorSubcoreMesh(core_axis_name="c", subcore_axis_name="s"),
        scratch_types=[
            pltpu.VMEM((tbl,), jnp.float32),
            pltpu.VMEM((_DG, bw), jnp.float32),
            pltpu.VMEM((_DG, bw), jnp.float32),
            pltpu.VMEM((bw,), jnp.int32),
            pltpu.VMEM((bw,), jnp.int32),
            pltpu.SemaphoreType.DMA,
            pltpu.SemaphoreType.DMA,
            pltpu.SemaphoreType.DMA,
            pltpu.SemaphoreType.DMA,
            pltpu.SemaphoreType.DMA,
            pltpu.SemaphoreType.DMA,
        ],
        compiler_params=_CP,
    )
    def sc_kernel(x_hbm, t_hbm, pe_hbm, o_hbm, pv, xb0, xb1, ib0, ib1,
                  sx0, sx1, si0, si1, so0, so1):
        wid = lax.axis_index("c") * _NS + lax.axis_index("s")
        g = wid % ng
        d0 = g * _DG
        b0 = (wid // ng) * bw

        pltpu.sync_copy(pe_hbm.at[pl.ds(g * tbl, tbl)], pv)

        xb = (xb0, xb1)
        ib = (ib0, ib1)
        sx = (sx0, sx1)
        si = (si0, si1)
        so = (so0, so1)

        def x_copy(h, p):
            return pltpu.make_async_copy(
                x_hbm.at[h, pl.ds(d0, _DG), pl.ds(b0, bw)], xb[p], sx[p])

        def i_copy(h, p):
            return pltpu.make_async_copy(
                t_hbm.at[h, pl.ds(b0, bw)], ib[p], si[p])

        def o_copy(h, p):
            return pltpu.make_async_copy(
                xb[p], o_hbm.at[h, pl.ds(d0, _DG), pl.ds(b0, bw)], so[p])

        def start_chunk(h, p):
            x_copy(h, p).start()
            i_copy(h, p).start()

        def wait_chunk(h, p):
            x_copy(h, p).wait()
            i_copy(h, p).wait()

        start_chunk(0, 0)

        @pl.loop(0, nh, step=2)
        def _outer(h0):
            for p in (0, 1):
                h = h0 + p
                q = 1 - p

                @pl.when(h + 1 < nh)
                def _prefetch():
                    @pl.when(h >= 1)
                    def _drain_prev_out():
                        o_copy(h - 1, q).wait()

                    start_chunk(h + 1, q)

                wait_chunk(h, p)

                @plsc.parallel_loop(0, bw // _LANES, unroll=4)
                def _j(j):
                    c0 = j * _LANES
                    tv = ib[p][pl.ds(c0, _LANES)]
                    fb = ((tv >> 7) << 10) + (tv & 127)
                    for dd in range(_DG):
                        gv = plsc.load_gather(
                            pv.at[pl.ds(dd * 128, (_TT - 1) * 1024 + 128)], [fb])
                        plsc.addupdate(xb[p].at[dd, pl.ds(c0, _LANES)], gv)

                o_copy(h, p).start()

        o_copy(nh - 2, 0).wait()
        o_copy(nh - 1, 1).wait()

    return sc_kernel(xP, tP, pe_flat)


@jax.jit
def kernel(x, t, pe):
    n_pe, d = pe.shape
    xP = jnp.transpose(x, (1, 2, 0))
    tP = jnp.transpose(t, (1, 0)).astype(jnp.int32)
    peT = jnp.transpose(pe, (1, 0))
    pePad = jnp.pad(peT, ((0, 0), (0, _TT * 128 - n_pe)))
    pe_flat = (
        pePad.reshape(d // _DG, _DG, _TT, 128)
        .transpose(0, 2, 1, 3)
        .reshape(-1)
    )
    outP = _sc_call(xP, tP, pe_flat)
    return jnp.transpose(outP, (2, 0, 1))


# speedup vs baseline: 13.2250x; 1.0046x over previous
import dataclasses
import functools

import jax
import jax.numpy as jnp
from jax import lax
from jax.experimental import pallas as pl
from jax.experimental.pallas import tpu as pltpu
from jax.experimental.pallas import tpu_sc as plsc

_LANES = 16
_NC = 2
_NS = 16
_NW = _NC * _NS
_DG = 8
_TT = 79

_CP = pltpu.CompilerParams()
if "needs_layout_passes" in pltpu.CompilerParams.__dataclass_fields__:
    _CP = dataclasses.replace(_CP, needs_layout_passes=False)


def _sc_call(xP, tP, pe_flat):
    nh, d, nb = xP.shape
    ng = d // _DG
    nbr = _NW // ng
    bw = nb // nbr
    tbl = _TT * _DG * 128

    @functools.partial(
        pl.kernel,
        out_type=jax.ShapeDtypeStruct((nh, d, nb), jnp.float32),
        mesh=plsc.VectorSubcoreMesh(core_axis_name="c", subcore_axis_name="s"),
        scratch_types=[
            pltpu.VMEM((tbl,), jnp.float32),
            pltpu.VMEM((_DG, bw), jnp.float32),
            pltpu.VMEM((_DG, bw), jnp.float32),
            pltpu.VMEM((bw,), jnp.int32),
            pltpu.VMEM((bw,), jnp.int32),
            pltpu.SemaphoreType.DMA,
            pltpu.SemaphoreType.DMA,
            pltpu.SemaphoreType.DMA,
            pltpu.SemaphoreType.DMA,
            pltpu.SemaphoreType.DMA,
            pltpu.SemaphoreType.DMA,
        ],
        compiler_params=_CP,
    )
    def sc_kernel(x_hbm, t_hbm, pe_hbm, o_hbm, pv, xb0, xb1, ib0, ib1,
                  sx0, sx1, si0, si1, so0, so1):
        wid = lax.axis_index("c") * _NS + lax.axis_index("s")
        g = wid % ng
        d0 = g * _DG
        b0 = (wid // ng) * bw

        pltpu.sync_copy(pe_hbm.at[pl.ds(g * tbl, tbl)], pv)

        xb = (xb0, xb1)
        ib = (ib0, ib1)
        sx = (sx0, sx1)
        si = (si0, si1)
        so = (so0, so1)

        def x_copy(h, p):
            return pltpu.make_async_copy(
                x_hbm.at[h, pl.ds(d0, _DG), pl.ds(b0, bw)], xb[p], sx[p])

        def i_copy(h, p):
            return pltpu.make_async_copy(
                t_hbm.at[h, pl.ds(b0, bw)], ib[p], si[p])

        def o_copy(h, p):
            return pltpu.make_async_copy(
                xb[p], o_hbm.at[h, pl.ds(d0, _DG), pl.ds(b0, bw)], so[p])

        def start_chunk(h, p):
            x_copy(h, p).start()
            i_copy(h, p).start()

        def wait_chunk(h, p):
            x_copy(h, p).wait()
            i_copy(h, p).wait()

        start_chunk(0, 0)

        @pl.loop(0, nh, step=2)
        def _outer(h0):
            for p in (0, 1):
                h = h0 + p
                q = 1 - p

                @pl.when(h + 1 < nh)
                def _prefetch():
                    @pl.when(h >= 1)
                    def _drain_prev_out():
                        o_copy(h - 1, q).wait()

                    start_chunk(h + 1, q)

                wait_chunk(h, p)

                @plsc.parallel_loop(0, bw // _LANES, unroll=8)
                def _j(j):
                    c0 = j * _LANES
                    tv = ib[p][pl.ds(c0, _LANES)]
                    fb = ((tv >> 7) << 10) + (tv & 127)
                    for dd in range(_DG):
                        gv = plsc.load_gather(
                            pv.at[pl.ds(dd * 128, (_TT - 1) * 1024 + 128)], [fb])
                        plsc.addupdate(xb[p].at[dd, pl.ds(c0, _LANES)], gv)

                o_copy(h, p).start()

        o_copy(nh - 2, 0).wait()
        o_copy(nh - 1, 1).wait()

    return sc_kernel(xP, tP, pe_flat)


@jax.jit
def kernel(x, t, pe):
    n_pe, d = pe.shape
    xP = jnp.transpose(x, (1, 2, 0))
    tP = jnp.transpose(t, (1, 0)).astype(jnp.int32)
    peT = jnp.transpose(pe, (1, 0))
    pePad = jnp.pad(peT, ((0, 0), (0, _TT * 128 - n_pe)))
    pe_flat = (
        pePad.reshape(d // _DG, _DG, _TT, 128)
        .transpose(0, 2, 1, 3)
        .reshape(-1)
    )
    outP = _sc_call(xP, tP, pe_flat)
    return jnp.transpose(outP, (2, 0, 1))


# speedup vs baseline: 14.4790x; 1.0948x over previous
import dataclasses
import functools

import jax
import jax.numpy as jnp
from jax import lax
from jax.experimental import pallas as pl
from jax.experimental.pallas import tpu as pltpu
from jax.experimental.pallas import tpu_sc as plsc

_LANES = 16
_NC = 2
_NS = 16
_NW = _NC * _NS
_DG = 8
_TT = 79
_HC = 2

_CP = pltpu.CompilerParams()
if "needs_layout_passes" in pltpu.CompilerParams.__dataclass_fields__:
    _CP = dataclasses.replace(_CP, needs_layout_passes=False)


def _sc_call(xP, tP, pe_flat):
    nh, d, nb = xP.shape
    ng = d // _DG
    nbr = _NW // ng
    bw = nb // nbr
    tbl = _TT * _DG * 128

    @functools.partial(
        pl.kernel,
        out_type=jax.ShapeDtypeStruct((nh, d, nb), jnp.float32),
        mesh=plsc.VectorSubcoreMesh(core_axis_name="c", subcore_axis_name="s"),
        scratch_types=[
            pltpu.VMEM((tbl,), jnp.float32),
            pltpu.VMEM((_HC, _DG, bw), jnp.float32),
            pltpu.VMEM((_HC, _DG, bw), jnp.float32),
            pltpu.VMEM((_HC, bw), jnp.int32),
            pltpu.VMEM((_HC, bw), jnp.int32),
            pltpu.SemaphoreType.DMA,
            pltpu.SemaphoreType.DMA,
            pltpu.SemaphoreType.DMA,
            pltpu.SemaphoreType.DMA,
            pltpu.SemaphoreType.DMA,
            pltpu.SemaphoreType.DMA,
        ],
        compiler_params=_CP,
    )
    def sc_kernel(x_hbm, t_hbm, pe_hbm, o_hbm, pv, xb0, xb1, ib0, ib1,
                  sx0, sx1, si0, si1, so0, so1):
        wid = lax.axis_index("c") * _NS + lax.axis_index("s")
        g = wid % ng
        d0 = g * _DG
        b0 = (wid // ng) * bw

        pltpu.sync_copy(pe_hbm.at[pl.ds(g * tbl, tbl)], pv)

        xb = (xb0, xb1)
        ib = (ib0, ib1)
        sx = (sx0, sx1)
        si = (si0, si1)
        so = (so0, so1)

        nch = nh // _HC

        def x_copy(c, p):
            return pltpu.make_async_copy(
                x_hbm.at[pl.ds(c * _HC, _HC), pl.ds(d0, _DG), pl.ds(b0, bw)],
                xb[p], sx[p])

        def i_copy(c, p):
            return pltpu.make_async_copy(
                t_hbm.at[pl.ds(c * _HC, _HC), pl.ds(b0, bw)], ib[p], si[p])

        def o_copy(c, p):
            return pltpu.make_async_copy(
                xb[p],
                o_hbm.at[pl.ds(c * _HC, _HC), pl.ds(d0, _DG), pl.ds(b0, bw)],
                so[p])

        def start_chunk(c, p):
            x_copy(c, p).start()
            i_copy(c, p).start()

        def wait_chunk(c, p):
            x_copy(c, p).wait()
            i_copy(c, p).wait()

        start_chunk(0, 0)

        @pl.loop(0, nch, step=2)
        def _outer(c0):
            for p in (0, 1):
                c = c0 + p
                q = 1 - p

                @pl.when(c + 1 < nch)
                def _prefetch():
                    @pl.when(c >= 1)
                    def _drain_prev_out():
                        o_copy(c - 1, q).wait()

                    start_chunk(c + 1, q)

                wait_chunk(c, p)

                for hh in range(_HC):
                    @plsc.parallel_loop(0, bw // _LANES, unroll=8)
                    def _j(j):
                        l0 = j * _LANES
                        tv = ib[p][hh, pl.ds(l0, _LANES)]
                        fb = ((tv >> 7) << 10) + (tv & 127)
                        for dd in range(_DG):
                            gv = plsc.load_gather(
                                pv.at[pl.ds(dd * 128, (_TT - 1) * 1024 + 128)],
                                [fb])
                            plsc.addupdate(
                                xb[p].at[hh, dd, pl.ds(l0, _LANES)], gv)

                o_copy(c, p).start()

        o_copy(nch - 2, 0).wait()
        o_copy(nch - 1, 1).wait()

    return sc_kernel(xP, tP, pe_flat)


@jax.jit
def kernel(x, t, pe):
    n_pe, d = pe.shape
    xP = jnp.transpose(x, (1, 2, 0))
    tP = jnp.transpose(t, (1, 0)).astype(jnp.int32)
    peT = jnp.transpose(pe, (1, 0))
    pePad = jnp.pad(peT, ((0, 0), (0, _TT * 128 - n_pe)))
    pe_flat = (
        pePad.reshape(d // _DG, _DG, _TT, 128)
        .transpose(0, 2, 1, 3)
        .reshape(-1)
    )
    outP = _sc_call(xP, tP, pe_flat)
    return jnp.transpose(outP, (2, 0, 1))
